# baseline (speedup 1.0000x reference)
import functools

import jax
import jax.numpy as jnp
from jax import lax
from jax.experimental import pallas as pl
from jax.experimental.pallas import tpu as pltpu
from jax.experimental.pallas import tpu_sc as plsc

D = 128
NC = 2
NS = 16
NW = NC * NS
CHUNK = 128
KBLK = 8
F0 = 0.5


def _pad_sizes(n_nodes, n_edges):
    quantum = NS * CHUNK * KBLK * 2
    epad = ((n_edges + quantum - 1) // quantum) * quantum
    stripe = ((n_nodes + 1 + NS - 1) // NS + 7) // 8 * 8
    nacc = stripe * NS
    return epad, nacc, stripe


def _split_chunks(epad):
    ct = epad // (NS * CHUNK)
    ch0 = int(ct * F0 + 0.5) // KBLK * KBLK
    ch0 = max(KBLK, min(ct - KBLK, ch0))
    return ch0, ct - ch0


@functools.lru_cache(maxsize=None)
def _make_sc_seg_sum(n_nodes, epad, nacc):
    ch0, ch1 = _split_chunks(epad)
    ep0, ep1 = ch0 * CHUNK, ch1 * CHUNK
    stripe = nacc // NS
    mesh = plsc.VectorSubcoreMesh(core_axis_name="c", subcore_axis_name="s")

    def body(sup, srcp, dstp, zeros, out, src_v, dst_v, rows_a, rows_b,
             acc, sem_a, sem_b, sem_sa, sem_sb, sem_d):
        c = lax.axis_index("c")
        s = lax.axis_index("s")
        cbase = pl.multiple_of(jnp.where(c == 0, s * ch0, NS * ch0 + s * ch1),
                               KBLK)
        mych = jnp.where(c == 0, ch0, ch1)
        @pl.when(c == 0)
        def _():
            pltpu.sync_copy(srcp.at[pl.ds(cbase * CHUNK, ep0)],
                            src_v.at[pl.ds(0, ep0)])

        @pl.when(c != 0)
        def _():
            pltpu.sync_copy(srcp.at[pl.ds(cbase * CHUNK, ep1)],
                            src_v.at[pl.ds(0, ep1)])

        pltpu.sync_copy(dstp.at[pl.ds(cbase, KBLK)], dst_v.at[0])

        @pl.when(KBLK < mych)
        def _():
            pltpu.async_copy(
                dstp.at[pl.ds(pl.multiple_of(cbase + KBLK, KBLK), KBLK)],
                dst_v.at[1], sem_d)

        pltpu.sync_copy(zeros.at[pl.ds(s * stripe, stripe)],
                        acc.at[pl.ds(s * stripe, stripe)])
        plsc.subcore_barrier()

        pltpu.async_copy(sup.at[src_v.at[pl.ds(0, CHUNK)]], rows_a, sem_a)
        pltpu.async_copy(sup.at[src_v.at[pl.ds(CHUNK, CHUNK)]], rows_b, sem_b)

        def pair(pp, carry):
            j = pp * 2
            blk = lax.div(j, KBLK)
            p = lax.rem(blk, 2)
            jl = lax.rem(j, KBLK)

            @pl.when((jl == 0) & (j > 0))
            def _():
                pltpu.make_async_copy(dstp.at[pl.ds(0, KBLK)], dst_v.at[p],
                                      sem_d).wait()

            @pl.when((jl == 0) & (j + KBLK < mych))
            def _():
                pltpu.async_copy(
                    dstp.at[pl.ds(pl.multiple_of(cbase + j + KBLK, KBLK),
                                  KBLK)],
                    dst_v.at[1 - p], sem_d)

            pltpu.make_async_copy(sup.at[src_v.at[pl.ds(0, CHUNK)]],
                                  rows_a, sem_a).wait()
            pltpu.async_copy(rows_a, acc.at[dst_v.at[p, jl]], sem_sa,
                             add=True)
            pltpu.make_async_copy(sup.at[src_v.at[pl.ds(0, CHUNK)]],
                                  rows_b, sem_b).wait()
            pltpu.async_copy(rows_b, acc.at[dst_v.at[p, jl + 1]], sem_sb,
                             add=True)

            pltpu.make_async_copy(rows_a, acc.at[dst_v.at[p, jl]],
                                  sem_sa).wait()

            @pl.when(j + 2 < mych)
            def _():
                pltpu.async_copy(
                    sup.at[src_v.at[pl.ds((j + 2) * CHUNK, CHUNK)]],
                    rows_a, sem_a)

            pltpu.make_async_copy(rows_b, acc.at[dst_v.at[p, jl + 1]],
                                  sem_sb).wait()

            @pl.when(j + 3 < mych)
            def _():
                pltpu.async_copy(
                    sup.at[src_v.at[pl.ds((j + 3) * CHUNK, CHUNK)]],
                    rows_b, sem_b)

            return carry

        lax.fori_loop(0, lax.div(mych, 2), pair, 0)
        plsc.subcore_barrier()
        pltpu.sync_copy(acc.at[pl.ds(s * stripe, stripe)],
                        out.at[c, pl.ds(s * stripe, stripe)])

    return pl.kernel(
        body,
        out_type=jax.ShapeDtypeStruct((NC, nacc, D), jnp.float32),
        mesh=mesh,
        scratch_types=[
            pltpu.VMEM((max(ep0, ep1),), jnp.int32),
            pltpu.VMEM((2, KBLK, CHUNK), jnp.int32),
            pltpu.VMEM((CHUNK, D), jnp.float32),
            pltpu.VMEM((CHUNK, D), jnp.float32),
            pltpu.VMEM_SHARED((nacc, D), jnp.float32),
            pltpu.SemaphoreType.DMA,
            pltpu.SemaphoreType.DMA,
            pltpu.SemaphoreType.DMA,
            pltpu.SemaphoreType.DMA,
            pltpu.SemaphoreType.DMA,
        ],
    )


def _tc_first(x, w, sw):
    n = x.shape[0]

    def body(x_ref, w_ref, sw_ref, sup_ref, self_ref):
        xv = x_ref[...]
        sup_ref[...] = jnp.dot(xv, w_ref[...],
                               preferred_element_type=jnp.float32)
        self_ref[...] = jnp.dot(xv, sw_ref[...],
                                preferred_element_type=jnp.float32)

    return pl.pallas_call(
        body,
        out_shape=(jax.ShapeDtypeStruct((n, D), jnp.float32),
                   jax.ShapeDtypeStruct((n, D), jnp.float32)),
    )(x, w, sw)


def _bn_relu(pre, g, b):
    m = jnp.mean(pre, axis=0, keepdims=True)
    v = jnp.mean((pre - m) ** 2, axis=0, keepdims=True)
    return jnp.maximum((pre - m) * jax.lax.rsqrt(v + 1e-5) * g + b, 0.0)


def _tc_mid(parts, selfp, g, b, w_next, sw_next):
    n = selfp.shape[0]

    def body(parts_ref, self_ref, g_ref, b_ref, w_ref, sw_ref,
             sup_ref, selfo_ref):
        pre = (parts_ref[0, :n, :] + parts_ref[1, :n, :] + self_ref[...])
        h = _bn_relu(pre, g_ref[...], b_ref[...])
        sup_ref[...] = jnp.dot(h, w_ref[...],
                               preferred_element_type=jnp.float32)
        selfo_ref[...] = jnp.dot(h, sw_ref[...],
                                 preferred_element_type=jnp.float32)

    return pl.pallas_call(
        body,
        out_shape=(jax.ShapeDtypeStruct((n, D), jnp.float32),
                   jax.ShapeDtypeStruct((n, D), jnp.float32)),
    )(parts, selfp, g.reshape(1, D), b.reshape(1, D), w_next, sw_next)


def _tc_last(parts, selfp, g, b, x):
    n = x.shape[0]

    def body(parts_ref, self_ref, g_ref, b_ref, x_ref, out_ref):
        pre = (parts_ref[0, :n, :] + parts_ref[1, :n, :] + self_ref[...])
        h = _bn_relu(pre, g_ref[...], b_ref[...])
        out_ref[...] = jnp.concatenate([h, x_ref[...]], axis=1)

    return pl.pallas_call(
        body,
        out_shape=jax.ShapeDtypeStruct((n, 2 * D), jnp.float32),
    )(parts, selfp, g.reshape(1, D), b.reshape(1, D), x)


def kernel(x, edge_index, W0, SW0, g0, b0, W1, SW1, g1, b1):
    n_nodes, _ = x.shape
    n_edges = edge_index.shape[1]
    epad, nacc, _ = _pad_sizes(n_nodes, n_edges)

    src = edge_index[0].astype(jnp.int32)
    dst = edge_index[1].astype(jnp.int32)
    pad = epad - n_edges
    pad_i = jnp.arange(pad, dtype=jnp.int32)
    srcp = jnp.concatenate([src, pad_i % n_nodes])
    dstp = jnp.concatenate([dst, n_nodes + pad_i % (nacc - n_nodes)])
    dstp = dstp.reshape(epad // CHUNK, CHUNK)
    zeros = jnp.zeros((nacc, D), jnp.float32)

    seg_sum = _make_sc_seg_sum(n_nodes, epad, nacc)

    sup0, self0 = _tc_first(x, W0, SW0)
    parts0 = seg_sum(sup0, srcp, dstp, zeros)
    sup1, self1 = _tc_mid(parts0, self0, g0, b0, W1, SW1)
    parts1 = seg_sum(sup1, srcp, dstp, zeros)
    return _tc_last(parts1, self1, g1, b1, x)


# speedup vs baseline: 1.0824x; 1.0824x over previous
import functools

import jax
import jax.numpy as jnp
from jax import lax
from jax.experimental import pallas as pl
from jax.experimental.pallas import tpu as pltpu
from jax.experimental.pallas import tpu_sc as plsc

D = 128
NC = 2
NS = 16
NW = NC * NS
CHUNK = 128
KBLK = 16
F0 = 0.5


def _pad_sizes(n_nodes, n_edges):
    quantum = NS * CHUNK * KBLK * 2
    epad = ((n_edges + quantum - 1) // quantum) * quantum
    stripe = ((n_nodes + 1 + NS - 1) // NS + 7) // 8 * 8
    nacc = stripe * NS
    return epad, nacc, stripe


def _split_chunks(epad):
    ct = epad // (NS * CHUNK)
    ch0 = int(ct * F0 + 0.5) // KBLK * KBLK
    ch0 = max(KBLK, min(ct - KBLK, ch0))
    return ch0, ct - ch0


@functools.lru_cache(maxsize=None)
def _make_sc_seg_sum(n_nodes, epad, nacc):
    ch0, ch1 = _split_chunks(epad)
    ep0, ep1 = ch0 * CHUNK, ch1 * CHUNK
    stripe = nacc // NS
    mesh = plsc.VectorSubcoreMesh(core_axis_name="c", subcore_axis_name="s")

    def body(sup, srcp, dstp, zeros, out, src_v, dst_v, rows_a, rows_b,
             acc, sem_a, sem_b):
        c = lax.axis_index("c")
        s = lax.axis_index("s")
        cbase = pl.multiple_of(jnp.where(c == 0, s * ch0, NS * ch0 + s * ch1),
                               KBLK)
        mych = jnp.where(c == 0, ch0, ch1)
        @pl.when(c == 0)
        def _():
            pltpu.sync_copy(srcp.at[pl.ds(cbase * CHUNK, ep0)],
                            src_v.at[pl.ds(0, ep0)])

        @pl.when(c != 0)
        def _():
            pltpu.sync_copy(srcp.at[pl.ds(cbase * CHUNK, ep1)],
                            src_v.at[pl.ds(0, ep1)])

        pltpu.sync_copy(dstp.at[pl.ds(cbase, KBLK)], dst_v)
        pltpu.sync_copy(zeros.at[pl.ds(s * stripe, stripe)],
                        acc.at[pl.ds(s * stripe, stripe)])
        plsc.subcore_barrier()

        pltpu.async_copy(sup.at[src_v.at[pl.ds(0, CHUNK)]], rows_a, sem_a)

        def pair(pp, carry):
            j = pp * 2
            jl = lax.rem(j, KBLK)
            pltpu.make_async_copy(sup.at[src_v.at[pl.ds(0, CHUNK)]],
                                  rows_a, sem_a).wait()
            pltpu.async_copy(
                sup.at[src_v.at[pl.ds((j + 1) * CHUNK, CHUNK)]],
                rows_b, sem_b)
            pltpu.sync_copy(rows_a, acc.at[dst_v.at[jl]], add=True)
            pltpu.make_async_copy(sup.at[src_v.at[pl.ds(0, CHUNK)]],
                                  rows_b, sem_b).wait()

            @pl.when(j + 2 < mych)
            def _():
                pltpu.async_copy(
                    sup.at[src_v.at[pl.ds((j + 2) * CHUNK, CHUNK)]],
                    rows_a, sem_a)

            pltpu.sync_copy(rows_b, acc.at[dst_v.at[jl + 1]], add=True)

            @pl.when((lax.rem(j + 2, KBLK) == 0) & (j + 2 < mych))
            def _():
                pltpu.sync_copy(
                    dstp.at[pl.ds(pl.multiple_of(cbase + j + 2, KBLK), KBLK)],
                    dst_v)

            return carry

        lax.fori_loop(0, lax.div(mych, 2), pair, 0)
        plsc.subcore_barrier()
        pltpu.sync_copy(acc.at[pl.ds(s * stripe, stripe)],
                        out.at[c, pl.ds(s * stripe, stripe)])

    return pl.kernel(
        body,
        out_type=jax.ShapeDtypeStruct((NC, nacc, D), jnp.float32),
        mesh=mesh,
        scratch_types=[
            pltpu.VMEM((max(ep0, ep1),), jnp.int32),
            pltpu.VMEM((KBLK, CHUNK), jnp.int32),
            pltpu.VMEM((CHUNK, D), jnp.float32),
            pltpu.VMEM((CHUNK, D), jnp.float32),
            pltpu.VMEM_SHARED((nacc, D), jnp.float32),
            pltpu.SemaphoreType.DMA,
            pltpu.SemaphoreType.DMA,
        ],
    )


def _tc_first(x, w, sw):
    n = x.shape[0]

    def body(x_ref, w_ref, sw_ref, sup_ref, self_ref):
        xv = x_ref[...]
        sup_ref[...] = jnp.dot(xv, w_ref[...],
                               preferred_element_type=jnp.float32)
        self_ref[...] = jnp.dot(xv, sw_ref[...],
                                preferred_element_type=jnp.float32)

    return pl.pallas_call(
        body,
        out_shape=(jax.ShapeDtypeStruct((n, D), jnp.float32),
                   jax.ShapeDtypeStruct((n, D), jnp.float32)),
    )(x, w, sw)


def _bn_relu(pre, g, b):
    m = jnp.mean(pre, axis=0, keepdims=True)
    v = jnp.mean((pre - m) ** 2, axis=0, keepdims=True)
    return jnp.maximum((pre - m) * jax.lax.rsqrt(v + 1e-5) * g + b, 0.0)


def _tc_mid(parts, selfp, g, b, w_next, sw_next):
    n = selfp.shape[0]

    def body(parts_ref, self_ref, g_ref, b_ref, w_ref, sw_ref,
             sup_ref, selfo_ref):
        pre = (parts_ref[0, :n, :] + parts_ref[1, :n, :] + self_ref[...])
        h = _bn_relu(pre, g_ref[...], b_ref[...])
        sup_ref[...] = jnp.dot(h, w_ref[...],
                               preferred_element_type=jnp.float32)
        selfo_ref[...] = jnp.dot(h, sw_ref[...],
                                 preferred_element_type=jnp.float32)

    return pl.pallas_call(
        body,
        out_shape=(jax.ShapeDtypeStruct((n, D), jnp.float32),
                   jax.ShapeDtypeStruct((n, D), jnp.float32)),
    )(parts, selfp, g.reshape(1, D), b.reshape(1, D), w_next, sw_next)


def _tc_last(parts, selfp, g, b, x):
    n = x.shape[0]

    def body(parts_ref, self_ref, g_ref, b_ref, x_ref, out_ref):
        pre = (parts_ref[0, :n, :] + parts_ref[1, :n, :] + self_ref[...])
        h = _bn_relu(pre, g_ref[...], b_ref[...])
        out_ref[...] = jnp.concatenate([h, x_ref[...]], axis=1)

    return pl.pallas_call(
        body,
        out_shape=jax.ShapeDtypeStruct((n, 2 * D), jnp.float32),
    )(parts, selfp, g.reshape(1, D), b.reshape(1, D), x)


def kernel(x, edge_index, W0, SW0, g0, b0, W1, SW1, g1, b1):
    n_nodes, _ = x.shape
    n_edges = edge_index.shape[1]
    epad, nacc, _ = _pad_sizes(n_nodes, n_edges)

    src = edge_index[0].astype(jnp.int32)
    dst = edge_index[1].astype(jnp.int32)
    pad = epad - n_edges
    pad_i = jnp.arange(pad, dtype=jnp.int32)
    srcp = jnp.concatenate([src, pad_i % n_nodes])
    dstp = jnp.concatenate([dst, n_nodes + pad_i % (nacc - n_nodes)])
    dstp = dstp.reshape(epad // CHUNK, CHUNK)
    zeros = jnp.zeros((nacc, D), jnp.float32)

    seg_sum = _make_sc_seg_sum(n_nodes, epad, nacc)

    sup0, self0 = _tc_first(x, W0, SW0)
    parts0 = seg_sum(sup0, srcp, dstp, zeros)
    sup1, self1 = _tc_mid(parts0, self0, g0, b0, W1, SW1)
    parts1 = seg_sum(sup1, srcp, dstp, zeros)
    return _tc_last(parts1, self1, g1, b1, x)


# speedup vs baseline: 1.1421x; 1.0552x over previous
import functools

import jax
import jax.numpy as jnp
from jax import lax
from jax.experimental import pallas as pl
from jax.experimental.pallas import tpu as pltpu
from jax.experimental.pallas import tpu_sc as plsc

D = 128
NC = 2
NS = 16
NW = NC * NS
CHUNK = 64
NBUF = 4
KBLK = 16
DBUF = 3


def _pad_sizes(n_nodes, n_edges):
    quantum = NW * CHUNK * KBLK
    epad = ((n_edges + quantum - 1) // quantum) * quantum
    stripe = ((n_nodes + 1 + NS - 1) // NS + 7) // 8 * 8
    nacc = stripe * NS
    return epad, nacc, stripe


@functools.lru_cache(maxsize=None)
def _make_sc_seg_sum(n_nodes, epad, nacc):
    ch = epad // (NW * CHUNK)
    ep = ch * CHUNK
    stripe = nacc // NS
    mesh = plsc.VectorSubcoreMesh(core_axis_name="c", subcore_axis_name="s")

    def body(sup, srcp, dstp, zeros, out, src_v, dst_v, rows_v,
             acc, sems, sem_d):
        c = lax.axis_index("c")
        s = lax.axis_index("s")
        wid = s * NC + c
        cbase = wid * ch
        pltpu.sync_copy(srcp.at[pl.ds(wid * ep, ep)], src_v)
        pltpu.sync_copy(dstp.at[pl.ds(cbase, KBLK)], dst_v.at[0])
        pltpu.async_copy(dstp.at[pl.ds(cbase + KBLK, KBLK)], dst_v.at[1],
                         sem_d)
        pltpu.sync_copy(zeros.at[pl.ds(s * stripe, stripe)],
                        acc.at[pl.ds(s * stripe, stripe)])
        plsc.subcore_barrier()

        pltpu.async_copy(sup.at[src_v.at[pl.ds(0, CHUNK)]], rows_v.at[0],
                         sems.at[0])
        pltpu.async_copy(sup.at[src_v.at[pl.ds(CHUNK, CHUNK)]], rows_v.at[1],
                         sems.at[1])

        def group(q, carry):
            c0 = q * NBUF

            @pl.when((lax.rem(q, KBLK // NBUF) == 0) & (q > 0))
            def _():
                pltpu.make_async_copy(dstp.at[pl.ds(0, KBLK)], dst_v.at[0],
                                      sem_d).wait()

            @pl.when((lax.rem(q, KBLK // NBUF) == 0) & (q > 0)
                     & (c0 + KBLK < ch))
            def _():
                blk = lax.div(c0, KBLK)
                pltpu.async_copy(
                    dstp.at[pl.ds(pl.multiple_of(cbase + c0 + KBLK, KBLK),
                                  KBLK)],
                    dst_v.at[lax.rem(blk + 1, DBUF)], sem_d)

            for i in range(NBUF):
                cc = c0 + i
                x = i
                y = (i + 2) % NBUF
                bp = lax.rem(lax.div(cc, KBLK), DBUF)
                jl = lax.rem(cc, KBLK)
                pltpu.make_async_copy(sup.at[src_v.at[pl.ds(0, CHUNK)]],
                                      rows_v.at[x], sems.at[x]).wait()
                pltpu.async_copy(rows_v.at[x], acc.at[dst_v.at[bp, jl]],
                                 sems.at[x], add=True)

                @pl.when(cc >= 2)
                def _():
                    pltpu.make_async_copy(
                        rows_v.at[y], acc.at[dst_v.at[bp, jl]],
                        sems.at[y]).wait()

                @pl.when(cc + 2 < ch)
                def _():
                    pltpu.async_copy(
                        sup.at[src_v.at[pl.ds((cc + 2) * CHUNK, CHUNK)]],
                        rows_v.at[y], sems.at[y])

            return carry

        lax.fori_loop(0, ch // NBUF, group, 0)
        pltpu.make_async_copy(sup.at[src_v.at[pl.ds(0, CHUNK)]],
                              rows_v.at[(ch - 2) % NBUF],
                              sems.at[(ch - 2) % NBUF]).wait()
        pltpu.make_async_copy(sup.at[src_v.at[pl.ds(0, CHUNK)]],
                              rows_v.at[(ch - 1) % NBUF],
                              sems.at[(ch - 1) % NBUF]).wait()
        plsc.subcore_barrier()
        pltpu.sync_copy(acc.at[pl.ds(s * stripe, stripe)],
                        out.at[c, pl.ds(s * stripe, stripe)])

    return pl.kernel(
        body,
        out_type=jax.ShapeDtypeStruct((NC, nacc, D), jnp.float32),
        mesh=mesh,
        scratch_types=[
            pltpu.VMEM((ep,), jnp.int32),
            pltpu.VMEM((DBUF, KBLK, CHUNK), jnp.int32),
            pltpu.VMEM((NBUF, CHUNK, D), jnp.float32),
            pltpu.VMEM_SHARED((nacc, D), jnp.float32),
            pltpu.SemaphoreType.DMA((NBUF,)),
            pltpu.SemaphoreType.DMA,
        ],
    )


def _tc_first(x, w, sw):
    n = x.shape[0]

    def body(x_ref, w_ref, sw_ref, sup_ref, self_ref):
        xv = x_ref[...]
        sup_ref[...] = jnp.dot(xv, w_ref[...],
                               preferred_element_type=jnp.float32)
        self_ref[...] = jnp.dot(xv, sw_ref[...],
                                preferred_element_type=jnp.float32)

    return pl.pallas_call(
        body,
        out_shape=(jax.ShapeDtypeStruct((n, D), jnp.float32),
                   jax.ShapeDtypeStruct((n, D), jnp.float32)),
    )(x, w, sw)


def _bn_relu(pre, g, b):
    m = jnp.mean(pre, axis=0, keepdims=True)
    v = jnp.mean((pre - m) ** 2, axis=0, keepdims=True)
    return jnp.maximum((pre - m) * jax.lax.rsqrt(v + 1e-5) * g + b, 0.0)


def _tc_mid(parts, selfp, g, b, w_next, sw_next):
    n = selfp.shape[0]

    def body(parts_ref, self_ref, g_ref, b_ref, w_ref, sw_ref,
             sup_ref, selfo_ref):
        pre = (parts_ref[0, :n, :] + parts_ref[1, :n, :] + self_ref[...])
        h = _bn_relu(pre, g_ref[...], b_ref[...])
        sup_ref[...] = jnp.dot(h, w_ref[...],
                               preferred_element_type=jnp.float32)
        selfo_ref[...] = jnp.dot(h, sw_ref[...],
                                 preferred_element_type=jnp.float32)

    return pl.pallas_call(
        body,
        out_shape=(jax.ShapeDtypeStruct((n, D), jnp.float32),
                   jax.ShapeDtypeStruct((n, D), jnp.float32)),
    )(parts, selfp, g.reshape(1, D), b.reshape(1, D), w_next, sw_next)


def _tc_last(parts, selfp, g, b, x):
    n = x.shape[0]

    def body(parts_ref, self_ref, g_ref, b_ref, x_ref, out_ref):
        pre = (parts_ref[0, :n, :] + parts_ref[1, :n, :] + self_ref[...])
        h = _bn_relu(pre, g_ref[...], b_ref[...])
        out_ref[...] = jnp.concatenate([h, x_ref[...]], axis=1)

    return pl.pallas_call(
        body,
        out_shape=jax.ShapeDtypeStruct((n, 2 * D), jnp.float32),
    )(parts, selfp, g.reshape(1, D), b.reshape(1, D), x)


def kernel(x, edge_index, W0, SW0, g0, b0, W1, SW1, g1, b1):
    n_nodes, _ = x.shape
    n_edges = edge_index.shape[1]
    epad, nacc, _ = _pad_sizes(n_nodes, n_edges)

    src = edge_index[0].astype(jnp.int32)
    dst = edge_index[1].astype(jnp.int32)
    pad = epad - n_edges
    pad_i = jnp.arange(pad, dtype=jnp.int32)
    srcp = jnp.concatenate([src, pad_i % n_nodes])
    dstp = jnp.concatenate([dst, n_nodes + pad_i % (nacc - n_nodes)])
    dstp = dstp.reshape(epad // CHUNK, CHUNK)
    zeros = jnp.zeros((nacc, D), jnp.float32)

    seg_sum = _make_sc_seg_sum(n_nodes, epad, nacc)

    sup0, self0 = _tc_first(x, W0, SW0)
    parts0 = seg_sum(sup0, srcp, dstp, zeros)
    sup1, self1 = _tc_mid(parts0, self0, g0, b0, W1, SW1)
    parts1 = seg_sum(sup1, srcp, dstp, zeros)
    return _tc_last(parts1, self1, g1, b1, x)
